# baseline (speedup 1.0000x reference)
import functools

import jax
import jax.numpy as jnp
from jax import lax
from jax.experimental import pallas as pl
from jax.experimental.pallas import tpu as pltpu
from jax.experimental.pallas import tpu_sc as plsc

N = 10000
E = 320000
D = 128
D_EDGE = 16
NLAYERS = 7
WCOLS = 16

NC = 2
NS = 16
NW = NC * NS

EP = E // NW
EB = 200
NEB = EP // EB
RPTN = 624
_CHUNKS = list(range(0, EB - 15, 16)) + ([EB - 16] if EB % 16 else [])
_PCH = 2000


def _mesh():
    return plsc.VectorSubcoreMesh(core_axis_name="c", subcore_axis_name="s",
                                  num_cores=NC, num_subcores=NS)


_NHALF = N // 2


def _sc_deg_body(w_hbm, dst_hbm, out_hbm, dbig_v, wblk_v, deg_v):
    c = lax.axis_index("c")
    s = lax.axis_index("s")
    wid = s * NC + c
    ebase = wid * EP

    iota16 = lax.iota(jnp.int32, 16)
    zero16 = (iota16 * 0).astype(jnp.float32)
    pltpu.sync_copy(dst_hbm.at[pl.ds(ebase, EP)], dbig_v)

    for half in range(2):
        lo = half * _NHALF

        def zloop(k, _):
            deg_v[pl.ds(16 * k, 16)] = zero16
            return 0
        lax.fori_loop(0, _NHALF * WCOLS // 16, zloop, 0)

        def body(b, _):
            off = ebase + b * EB
            pltpu.sync_copy(w_hbm.at[pl.ds(off * WCOLS, EB * WCOLS)], wblk_v)

            def row(r, _):
                dsp = plsc.load_gather(dbig_v, [iota16 * 0 + (b * EB + r)])
                inb = (dsp >= lo) & (dsp < lo + _NHALF)
                ridx = jnp.where(inb, dsp - lo, 0) * WCOLS + iota16
                plsc.addupdate_scatter(deg_v, [ridx],
                                       wblk_v[pl.ds(r * WCOLS, 16)], mask=inb)
                return 0
            lax.fori_loop(0, EB, row, 0)
            return 0
        lax.fori_loop(0, NEB, body, 0)
        pltpu.sync_copy(deg_v, out_hbm.at[wid, pl.ds(lo * WCOLS, _NHALF * WCOLS)])


@functools.lru_cache(maxsize=None)
def _sc_deg_fn():
    return pl.kernel(
        _sc_deg_body,
        out_type=jax.ShapeDtypeStruct((NW, N * WCOLS), jnp.float32),
        mesh=_mesh(),
        compiler_params=pltpu.CompilerParams(needs_layout_passes=False),
        scratch_types=[
            pltpu.VMEM((EP,), jnp.int32),
            pltpu.VMEM((EB * WCOLS,), jnp.float32),
            pltpu.VMEM((_NHALF * WCOLS,), jnp.float32),
        ],
    )


def _sc_deg(w_all, dst):
    return _sc_deg_fn()(w_all.reshape(E * WCOLS), dst)


def _sc_layer_body(h_hbm, src_hbm, dst_hbm, wcol_hbm, out_hbm,
                   sidx_v, didx_v, dbig_v, wbig_v,
                   rows_v, zbuf_v, acc_sh, sem):
    c = lax.axis_index("c")
    s = lax.axis_index("s")
    wid = s * NC + c
    nbase = s * RPTN

    iota16 = lax.iota(jnp.int32, 16)
    zero16 = (iota16 * 0).astype(jnp.float32)
    for i in range(16):
        for j in range(D // 16):
            zbuf_v[i, pl.ds(j * 16, 16)] = zero16

    def zloop(k, _):
        pltpu.sync_copy(zbuf_v, acc_sh.at[pl.ds(nbase + 16 * k, 16)])
        return 0
    lax.fori_loop(0, RPTN // 16, zloop, 0)

    @pl.when(s == NS - 1)
    def _():
        pltpu.sync_copy(zbuf_v, acc_sh.at[pl.ds(N - 16, 16)])
    plsc.subcore_barrier()

    ebase = wid * EP

    pltpu.sync_copy(wcol_hbm.at[pl.ds(ebase, EP)], wbig_v)
    pltpu.sync_copy(dst_hbm.at[pl.ds(ebase, EP)], dbig_v)

    def body(b, _):
        o = b * EB
        pltpu.sync_copy(src_hbm.at[pl.ds(ebase + o, EB)], sidx_v)
        for k in _CHUNKS:
            didx_v[pl.ds(k, 16)] = dbig_v[pl.ds(o + k, 16)]
        pltpu.async_copy(h_hbm.at[sidx_v], rows_v, sem).wait()

        def row(r, _):
            sp = plsc.load_gather(wbig_v, [iota16 * 0 + (o + r)])
            for j in range(D // 16):
                sl = pl.ds(j * 16, 16)
                rows_v[r, sl] = rows_v[r, sl] * sp
            return 0
        lax.fori_loop(0, EB, row, 0)
        pltpu.sync_copy(rows_v, acc_sh.at[didx_v], add=True)
        return 0
    lax.fori_loop(0, NEB, body, 0)
    plsc.subcore_barrier()

    @pl.when(s < NS - 1)
    def _():
        pltpu.sync_copy(acc_sh.at[pl.ds(nbase, RPTN)],
                        out_hbm.at[c, pl.ds(nbase, RPTN)])

    @pl.when(s == NS - 1)
    def _():
        pltpu.sync_copy(acc_sh.at[pl.ds(nbase, N - (NS - 1) * RPTN)],
                        out_hbm.at[c, pl.ds(nbase, N - (NS - 1) * RPTN)])


@functools.lru_cache(maxsize=None)
def _sc_layer_fn():
    return pl.kernel(
        _sc_layer_body,
        out_type=jax.ShapeDtypeStruct((NC, N, D), jnp.float32),
        mesh=_mesh(),
        compiler_params=pltpu.CompilerParams(needs_layout_passes=False),
        scratch_types=[
            pltpu.VMEM((EB,), jnp.int32),
            pltpu.VMEM((EB,), jnp.int32),
            pltpu.VMEM((EP,), jnp.int32),
            pltpu.VMEM((EP,), jnp.float32),
            pltpu.VMEM((EB, D), jnp.float32),
            pltpu.VMEM((16, D), jnp.float32),
            pltpu.VMEM_SHARED((N, D), jnp.float32),
            pltpu.SemaphoreType.DMA,
        ],
    )


def _sc_layer(hs, src, dst, wcol):
    return _sc_layer_fn()(hs, src, dst, wcol)


_EBLK = 8000


def _edge_mlp_body(ea_ref, w1_ref, b1_ref, w2_ref, b2_ref, out_ref):
    hid = jnp.maximum(
        jnp.dot(ea_ref[...], w1_ref[...], preferred_element_type=jnp.float32)
        + b1_ref[...], 0.0)
    z = jnp.dot(hid, w2_ref[...], preferred_element_type=jnp.float32) + b2_ref[...]
    out_ref[...] = jax.nn.sigmoid(z)


def _edge_mlp(edge_attr, w1cat, b1cat, w2blk, b2v):
    nh = NLAYERS * D_EDGE
    return pl.pallas_call(
        _edge_mlp_body,
        grid=(E // _EBLK,),
        in_specs=[
            pl.BlockSpec((_EBLK, D_EDGE), lambda i: (i, 0)),
            pl.BlockSpec((D_EDGE, nh), lambda i: (0, 0)),
            pl.BlockSpec((1, nh), lambda i: (0, 0)),
            pl.BlockSpec((nh, WCOLS), lambda i: (0, 0)),
            pl.BlockSpec((1, WCOLS), lambda i: (0, 0)),
        ],
        out_specs=pl.BlockSpec((_EBLK, WCOLS), lambda i: (i, 0)),
        out_shape=jax.ShapeDtypeStruct((E, WCOLS), jnp.float32),
    )(edge_attr, w1cat, b1cat, w2blk, b2v)


def _dinv_body(degp_ref, dinv_ref):
    d = jnp.sum(degp_ref[...], axis=0, keepdims=True) + 1.0
    dinv_ref[...] = lax.rsqrt(d)


def _dinv(deg_parts):
    flat = deg_parts.reshape(NW, N * WCOLS)
    out = pl.pallas_call(
        _dinv_body,
        out_shape=jax.ShapeDtypeStruct((1, N * WCOLS), jnp.float32),
    )(flat)
    return out.reshape(N, WCOLS)


def _mm0_body(x_ref, w_ref, dinv_ref, out_ref):
    out_ref[...] = dinv_ref[:, 0:1] * jnp.dot(
        x_ref[...], w_ref[...], preferred_element_type=jnp.float32)


def _mm0(x, w, dinv):
    return pl.pallas_call(
        _mm0_body,
        out_shape=jax.ShapeDtypeStruct((N, D), jnp.float32),
    )(x, w, dinv)


def _layer_tc_body(col, nskip, keep, refs):
    parts_ref = refs[0]
    hs_ref = refs[1]
    dinv_ref = refs[2]
    b_ref = refs[3]
    w_ref = refs[4]
    skip_refs = refs[5:5 + nskip]
    out_idx = 5 + nskip
    if keep:
        xk_ref = refs[out_idx]
        hn_ref = refs[out_idx + 1]
    else:
        hn_ref = refs[out_idx]

    xcur = (dinv_ref[:, col:col + 1]
            * (parts_ref[0] + parts_ref[1] + hs_ref[...]) + b_ref[...])
    for sref in skip_refs:
        xcur = xcur + sref[...]
    if keep:
        xk_ref[...] = xcur
    m = jnp.mean(xcur, axis=0, keepdims=True)
    cen = xcur - m
    v = jnp.mean(cen * cen, axis=0, keepdims=True)
    t = jnp.maximum(cen * lax.rsqrt(v + 1e-5), 0.0)
    hn_ref[...] = dinv_ref[:, col + 1:col + 2] * jnp.dot(
        t, w_ref[...], preferred_element_type=jnp.float32)


def _layer_tc(col, nskip, keep, parts, hs, dinv, b, wnext, skips):
    out_shape = [jax.ShapeDtypeStruct((N, D), jnp.float32)]
    if keep:
        out_shape = [jax.ShapeDtypeStruct((N, D), jnp.float32)] + out_shape
    body = lambda *refs: _layer_tc_body(col, nskip, keep, refs)
    res = pl.pallas_call(
        body,
        out_shape=out_shape,
    )(parts, hs, dinv, b, wnext, *skips)
    if keep:
        return res[0], res[1]
    return None, res[0]


def _final_body(parts_ref, hs_ref, dinv_ref, b_ref, x0_ref, x1_ref, x2_ref,
                batch_ref, linw_ref, linb_ref, out_ref, pool_ref):
    col = NLAYERS - 1
    xf = (dinv_ref[:, col:col + 1]
          * (parts_ref[0] + parts_ref[1] + hs_ref[...]) + b_ref[...]
          + x0_ref[...] + x1_ref[...] + x2_ref[...])
    x3 = jnp.maximum(xf, 0.0)
    bvec = batch_ref[...]

    def seg(g, _):
        msk = bvec == g
        mx = jnp.max(jnp.where(msk, x3, -jnp.inf), axis=0, keepdims=True)
        pool_ref[pl.ds(g, 1), :] = mx
        return 0
    lax.fori_loop(0, 64, seg, 0)
    out_ref[...] = (jnp.dot(pool_ref[...], linw_ref[...],
                            preferred_element_type=jnp.float32)
                    + linb_ref[...])


def _final_tc(parts, hs, dinv, b, x0, x1, x2, batch2d, lin_W, lin_b):
    return pl.pallas_call(
        _final_body,
        out_shape=jax.ShapeDtypeStruct((64, 16), jnp.float32),
        scratch_shapes=[pltpu.VMEM((64, D), jnp.float32)],
    )(parts, hs, dinv, b, x0, x1, x2, batch2d, lin_W, lin_b)


def kernel(x, edge_index, batch, dropout, edge_attr, device,
           c1_Wlin, c1_b, c1_W1, c1_b1, c1_W2, c1_b2,
           h1_Wlin, h1_b, h1_W1, h1_b1, h1_W2, h1_b2,
           h2_Wlin, h2_b, h2_W1, h2_b1, h2_W2, h2_b2,
           h3_Wlin, h3_b, h3_W1, h3_b1, h3_W2, h3_b2,
           lin_W, lin_b):
    del dropout, device

    layers = [(c1_Wlin, c1_b, c1_W1, c1_b1, c1_W2, c1_b2)]
    for (Wl, bl, W1l, b1l, W2l, b2l) in ((h1_Wlin, h1_b, h1_W1, h1_b1, h1_W2, h1_b2),
                                         (h2_Wlin, h2_b, h2_W1, h2_b1, h2_W2, h2_b2),
                                         (h3_Wlin, h3_b, h3_W1, h3_b1, h3_W2, h3_b2)):
        for i in range(2):
            layers.append((Wl[i], bl[i], W1l[i], b1l[i], W2l[i], b2l[i]))

    nh = NLAYERS * D_EDGE
    w1cat = jnp.concatenate([p[2] for p in layers], axis=1)
    b1cat = jnp.concatenate([p[3] for p in layers]).reshape(1, nh)
    w2blk = jnp.zeros((nh, WCOLS), jnp.float32)
    b2v = jnp.zeros((WCOLS,), jnp.float32)
    for l, p in enumerate(layers):
        w2blk = w2blk.at[l * D_EDGE:(l + 1) * D_EDGE, l].set(p[4][:, 0])
        b2v = b2v.at[l].set(p[5][0])
    b2v = b2v.reshape(1, WCOLS)

    src = edge_index[0]
    dst = edge_index[1]

    w_all = _edge_mlp(edge_attr, w1cat, b1cat, w2blk, b2v)
    deg_parts = _sc_deg(w_all, dst)
    dinv = _dinv(deg_parts)

    hs = _mm0(x, layers[0][0], dinv)

    keeps = []
    for l in range(NLAYERS):
        wcol = w_all[:, l]
        parts = _sc_layer(hs, src, dst, wcol)
        b_l = layers[l][1].reshape(1, D)
        if l == NLAYERS - 1:
            batch2d = batch.reshape(N, 1)
            return _final_tc(parts, hs, dinv, b_l, keeps[0], keeps[1],
                             keeps[2], batch2d, lin_W, lin_b)
        keep = l in (0, 2, 4)
        nskip = {0: 0, 1: 0, 2: 1, 3: 0, 4: 2, 5: 0}[l]
        skips = keeps[:nskip]
        wnext = layers[l + 1][0]
        xk, hs = _layer_tc(l, nskip, keep, parts, hs, dinv, b_l, wnext, skips)
        if keep:
            keeps.append(xk)


# speedup vs baseline: 1.0001x; 1.0001x over previous
import functools

import jax
import jax.numpy as jnp
from jax import lax
from jax.experimental import pallas as pl
from jax.experimental.pallas import tpu as pltpu
from jax.experimental.pallas import tpu_sc as plsc

N = 10000
E = 320000
D = 128
D_EDGE = 16
NLAYERS = 7
WCOLS = 16

NC = 2
NS = 16
NW = NC * NS

EP = E // NW
EB = 200
NEB = EP // EB
RPTN = 624
_CHUNKS = list(range(0, EB - 15, 16)) + ([EB - 16] if EB % 16 else [])


def _mesh():
    return plsc.VectorSubcoreMesh(core_axis_name="c", subcore_axis_name="s",
                                  num_cores=NC, num_subcores=NS)


_NHALF = N // 2


def _sc_deg_body(w_hbm, dst_hbm, out_hbm, dbig_v, wblk_v, deg_v):
    c = lax.axis_index("c")
    s = lax.axis_index("s")
    wid = s * NC + c
    ebase = wid * EP

    iota16 = lax.iota(jnp.int32, 16)
    zero16 = (iota16 * 0).astype(jnp.float32)
    pltpu.sync_copy(dst_hbm.at[pl.ds(ebase, EP)], dbig_v)

    for half in range(2):
        lo = half * _NHALF

        def zloop(k, _):
            deg_v[pl.ds(16 * k, 16)] = zero16
            return 0
        lax.fori_loop(0, _NHALF * WCOLS // 16, zloop, 0)

        def body(b, _):
            off = ebase + b * EB
            pltpu.sync_copy(w_hbm.at[pl.ds(off * WCOLS, EB * WCOLS)], wblk_v)

            def row(r, _):
                dsp = plsc.load_gather(dbig_v, [iota16 * 0 + (b * EB + r)])
                inb = (dsp >= lo) & (dsp < lo + _NHALF)
                ridx = jnp.where(inb, dsp - lo, 0) * WCOLS + iota16
                plsc.addupdate_scatter(deg_v, [ridx],
                                       wblk_v[pl.ds(r * WCOLS, 16)], mask=inb)
                return 0
            lax.fori_loop(0, EB, row, 0)
            return 0
        lax.fori_loop(0, NEB, body, 0)
        pltpu.sync_copy(deg_v, out_hbm.at[wid, pl.ds(lo * WCOLS, _NHALF * WCOLS)])


@functools.lru_cache(maxsize=None)
def _sc_deg_fn():
    return pl.kernel(
        _sc_deg_body,
        out_type=jax.ShapeDtypeStruct((NW, N * WCOLS), jnp.float32),
        mesh=_mesh(),
        compiler_params=pltpu.CompilerParams(needs_layout_passes=False),
        scratch_types=[
            pltpu.VMEM((EP,), jnp.int32),
            pltpu.VMEM((EB * WCOLS,), jnp.float32),
            pltpu.VMEM((_NHALF * WCOLS,), jnp.float32),
        ],
    )


def _sc_deg(w_all, dst):
    return _sc_deg_fn()(w_all.reshape(E * WCOLS), dst)


def _sc_layer_body(h_hbm, src_hbm, dst_hbm, wcol_hbm, out_hbm,
                   sidx_v, didx_v, dbig_v, wbig_v,
                   rows_v, zbuf_v, acc_sh, sem):
    c = lax.axis_index("c")
    s = lax.axis_index("s")
    wid = s * NC + c
    nbase = s * RPTN

    iota16 = lax.iota(jnp.int32, 16)
    zero16 = (iota16 * 0).astype(jnp.float32)
    for i in range(16):
        for j in range(D // 16):
            zbuf_v[i, pl.ds(j * 16, 16)] = zero16

    def zloop(k, _):
        pltpu.sync_copy(zbuf_v, acc_sh.at[pl.ds(nbase + 16 * k, 16)])
        return 0
    lax.fori_loop(0, RPTN // 16, zloop, 0)

    @pl.when(s == NS - 1)
    def _():
        pltpu.sync_copy(zbuf_v, acc_sh.at[pl.ds(N - 16, 16)])
    plsc.subcore_barrier()

    ebase = wid * EP

    pltpu.sync_copy(wcol_hbm.at[pl.ds(ebase, EP)], wbig_v)
    pltpu.sync_copy(dst_hbm.at[pl.ds(ebase, EP)], dbig_v)

    def body(b, _):
        o = b * EB
        pltpu.sync_copy(src_hbm.at[pl.ds(ebase + o, EB)], sidx_v)
        for k in _CHUNKS:
            didx_v[pl.ds(k, 16)] = dbig_v[pl.ds(o + k, 16)]
        pltpu.async_copy(h_hbm.at[sidx_v], rows_v, sem).wait()

        def row(r, _):
            sp = plsc.load_gather(wbig_v, [iota16 * 0 + (o + r)])
            for j in range(D // 16):
                sl = pl.ds(j * 16, 16)
                rows_v[r, sl] = rows_v[r, sl] * sp
            return 0
        lax.fori_loop(0, EB, row, 0)
        pltpu.sync_copy(rows_v, acc_sh.at[didx_v], add=True)
        return 0
    lax.fori_loop(0, NEB, body, 0)
    plsc.subcore_barrier()

    @pl.when(s < NS - 1)
    def _():
        pltpu.sync_copy(acc_sh.at[pl.ds(nbase, RPTN)],
                        out_hbm.at[c, pl.ds(nbase, RPTN)])

    @pl.when(s == NS - 1)
    def _():
        pltpu.sync_copy(acc_sh.at[pl.ds(nbase, N - (NS - 1) * RPTN)],
                        out_hbm.at[c, pl.ds(nbase, N - (NS - 1) * RPTN)])


@functools.lru_cache(maxsize=None)
def _sc_layer_fn():
    return pl.kernel(
        _sc_layer_body,
        out_type=jax.ShapeDtypeStruct((NC, N, D), jnp.float32),
        mesh=_mesh(),
        compiler_params=pltpu.CompilerParams(needs_layout_passes=False),
        scratch_types=[
            pltpu.VMEM((EB,), jnp.int32),
            pltpu.VMEM((EB,), jnp.int32),
            pltpu.VMEM((EP,), jnp.int32),
            pltpu.VMEM((EP,), jnp.float32),
            pltpu.VMEM((EB, D), jnp.float32),
            pltpu.VMEM((16, D), jnp.float32),
            pltpu.VMEM_SHARED((N, D), jnp.float32),
            pltpu.SemaphoreType.DMA,
        ],
    )


def _sc_layer(hs, src, dst, wcol):
    return _sc_layer_fn()(hs, src, dst, wcol)


_EBLK = 8000


def _edge_mlp_body(ea_ref, w1_ref, b1_ref, w2_ref, b2_ref, out_ref):
    hid = jnp.maximum(
        jnp.dot(ea_ref[...], w1_ref[...], preferred_element_type=jnp.float32)
        + b1_ref[...], 0.0)
    z = jnp.dot(hid, w2_ref[...], preferred_element_type=jnp.float32) + b2_ref[...]
    out_ref[...] = jax.nn.sigmoid(z)


def _edge_mlp(edge_attr, w1cat, b1cat, w2blk, b2v):
    nh = NLAYERS * D_EDGE
    return pl.pallas_call(
        _edge_mlp_body,
        grid=(E // _EBLK,),
        in_specs=[
            pl.BlockSpec((_EBLK, D_EDGE), lambda i: (i, 0)),
            pl.BlockSpec((D_EDGE, nh), lambda i: (0, 0)),
            pl.BlockSpec((1, nh), lambda i: (0, 0)),
            pl.BlockSpec((nh, WCOLS), lambda i: (0, 0)),
            pl.BlockSpec((1, WCOLS), lambda i: (0, 0)),
        ],
        out_specs=pl.BlockSpec((_EBLK, WCOLS), lambda i: (i, 0)),
        out_shape=jax.ShapeDtypeStruct((E, WCOLS), jnp.float32),
    )(edge_attr, w1cat, b1cat, w2blk, b2v)


def _dinv_body(degp_ref, dinv_ref):
    d = jnp.sum(degp_ref[...], axis=0, keepdims=True) + 1.0
    dinv_ref[...] = lax.rsqrt(d)


def _dinv(deg_parts):
    flat = deg_parts.reshape(NW, N * WCOLS)
    out = pl.pallas_call(
        _dinv_body,
        out_shape=jax.ShapeDtypeStruct((1, N * WCOLS), jnp.float32),
    )(flat)
    return out.reshape(N, WCOLS)


def _mm0_body(x_ref, w_ref, dinv_ref, out_ref):
    out_ref[...] = dinv_ref[:, 0:1] * jnp.dot(
        x_ref[...], w_ref[...], preferred_element_type=jnp.float32)


def _mm0(x, w, dinv):
    return pl.pallas_call(
        _mm0_body,
        out_shape=jax.ShapeDtypeStruct((N, D), jnp.float32),
    )(x, w, dinv)


def _layer_tc_body(col, nskip, keep, refs):
    parts_ref = refs[0]
    hs_ref = refs[1]
    dinv_ref = refs[2]
    b_ref = refs[3]
    w_ref = refs[4]
    skip_refs = refs[5:5 + nskip]
    out_idx = 5 + nskip
    if keep:
        xk_ref = refs[out_idx]
        hn_ref = refs[out_idx + 1]
    else:
        hn_ref = refs[out_idx]

    xcur = (dinv_ref[:, col:col + 1]
            * (parts_ref[0] + parts_ref[1] + hs_ref[...]) + b_ref[...])
    for sref in skip_refs:
        xcur = xcur + sref[...]
    if keep:
        xk_ref[...] = xcur
    m = jnp.mean(xcur, axis=0, keepdims=True)
    cen = xcur - m
    v = jnp.mean(cen * cen, axis=0, keepdims=True)
    t = jnp.maximum(cen * lax.rsqrt(v + 1e-5), 0.0)
    hn_ref[...] = dinv_ref[:, col + 1:col + 2] * jnp.dot(
        t, w_ref[...], preferred_element_type=jnp.float32)


def _layer_tc(col, nskip, keep, parts, hs, dinv, b, wnext, skips):
    out_shape = [jax.ShapeDtypeStruct((N, D), jnp.float32)]
    if keep:
        out_shape = [jax.ShapeDtypeStruct((N, D), jnp.float32)] + out_shape
    body = lambda *refs: _layer_tc_body(col, nskip, keep, refs)
    res = pl.pallas_call(
        body,
        out_shape=out_shape,
    )(parts, hs, dinv, b, wnext, *skips)
    if keep:
        return res[0], res[1]
    return None, res[0]


def _final_body(parts_ref, hs_ref, dinv_ref, b_ref, x0_ref, x1_ref, x2_ref,
                batch_ref, linw_ref, linb_ref, out_ref, pool_ref):
    col = NLAYERS - 1
    xf = (dinv_ref[:, col:col + 1]
          * (parts_ref[0] + parts_ref[1] + hs_ref[...]) + b_ref[...]
          + x0_ref[...] + x1_ref[...] + x2_ref[...])
    x3 = jnp.maximum(xf, 0.0)
    bvec = batch_ref[...]

    def seg(g, _):
        msk = bvec == g
        mx = jnp.max(jnp.where(msk, x3, -jnp.inf), axis=0, keepdims=True)
        pool_ref[pl.ds(g, 1), :] = mx
        return 0
    lax.fori_loop(0, 64, seg, 0)
    out_ref[...] = (jnp.dot(pool_ref[...], linw_ref[...],
                            preferred_element_type=jnp.float32)
                    + linb_ref[...])


def _final_tc(parts, hs, dinv, b, x0, x1, x2, batch2d, lin_W, lin_b):
    return pl.pallas_call(
        _final_body,
        out_shape=jax.ShapeDtypeStruct((64, 16), jnp.float32),
        scratch_shapes=[pltpu.VMEM((64, D), jnp.float32)],
    )(parts, hs, dinv, b, x0, x1, x2, batch2d, lin_W, lin_b)


def kernel(x, edge_index, batch, dropout, edge_attr, device,
           c1_Wlin, c1_b, c1_W1, c1_b1, c1_W2, c1_b2,
           h1_Wlin, h1_b, h1_W1, h1_b1, h1_W2, h1_b2,
           h2_Wlin, h2_b, h2_W1, h2_b1, h2_W2, h2_b2,
           h3_Wlin, h3_b, h3_W1, h3_b1, h3_W2, h3_b2,
           lin_W, lin_b):
    del dropout, device

    layers = [(c1_Wlin, c1_b, c1_W1, c1_b1, c1_W2, c1_b2)]
    for (Wl, bl, W1l, b1l, W2l, b2l) in ((h1_Wlin, h1_b, h1_W1, h1_b1, h1_W2, h1_b2),
                                         (h2_Wlin, h2_b, h2_W1, h2_b1, h2_W2, h2_b2),
                                         (h3_Wlin, h3_b, h3_W1, h3_b1, h3_W2, h3_b2)):
        for i in range(2):
            layers.append((Wl[i], bl[i], W1l[i], b1l[i], W2l[i], b2l[i]))

    nh = NLAYERS * D_EDGE
    w1cat = jnp.concatenate([p[2] for p in layers], axis=1)
    b1cat = jnp.concatenate([p[3] for p in layers]).reshape(1, nh)
    w2blk = jnp.zeros((nh, WCOLS), jnp.float32)
    b2v = jnp.zeros((WCOLS,), jnp.float32)
    for l, p in enumerate(layers):
        w2blk = w2blk.at[l * D_EDGE:(l + 1) * D_EDGE, l].set(p[4][:, 0])
        b2v = b2v.at[l].set(p[5][0])
    b2v = b2v.reshape(1, WCOLS)

    src = edge_index[0]
    dst = edge_index[1]

    w_all = _edge_mlp(edge_attr, w1cat, b1cat, w2blk, b2v)
    deg_parts = _sc_deg(w_all, dst)
    dinv = _dinv(deg_parts)

    hs = _mm0(x, layers[0][0], dinv)

    keeps = []
    for l in range(NLAYERS):
        wcol = w_all[:, l]
        parts = _sc_layer(hs, src, dst, wcol)
        b_l = layers[l][1].reshape(1, D)
        if l == NLAYERS - 1:
            batch2d = batch.reshape(N, 1)
            return _final_tc(parts, hs, dinv, b_l, keeps[0], keeps[1],
                             keeps[2], batch2d, lin_W, lin_b)
        keep = l in (0, 2, 4)
        nskip = {0: 0, 1: 0, 2: 1, 3: 0, 4: 2, 5: 0}[l]
        skips = keeps[:nskip]
        wnext = layers[l + 1][0]
        xk, hs = _layer_tc(l, nskip, keep, parts, hs, dinv, b_l, wnext, skips)
        if keep:
            keeps.append(xk)
